# Initial kernel scaffold; baseline (speedup 1.0000x reference)
import jax
import jax.numpy as jnp
from jax.experimental import pallas as pl


def kernel(src, indices):
    raise NotImplementedError("write your pallas kernel here")


# speedup vs baseline: 4.8953x; 4.8953x over previous
import functools

import jax
import jax.numpy as jnp
from jax import lax
from jax.experimental import pallas as pl
from jax.experimental.pallas import tpu as pltpu
from jax.experimental.pallas import tpu_sc as plsc

NUM_NODES = 10000
NUM_EDGES = 320000
FEAT = 128

NC = 2
NS = 16
NW = NC * NS

EDGES_PER_W = NUM_EDGES // NW
CHUNK = 200
NCHUNKS = EDGES_PER_W // CHUNK
SUB = 100
KSUB = CHUNK // SUB
ROWS_A = 632
ROWS_LAST = NUM_NODES - ROWS_A * (NS - 1)

_mesh = plsc.VectorSubcoreMesh(core_axis_name="c", subcore_axis_name="s")


def _split(count):
    out = [CHUNK] * (count // CHUNK)
    if count % CHUNK:
        out.append(count % CHUNK)
    return out


@functools.partial(
    pl.kernel,
    out_type=jax.ShapeDtypeStruct((NC, NUM_NODES, FEAT), jnp.float32),
    mesh=_mesh,
    scratch_types=[
        pltpu.VMEM_SHARED((NUM_NODES, FEAT), jnp.float32),
        pltpu.VMEM((CHUNK, FEAT), jnp.float32),
        pltpu.VMEM((KSUB, SUB), jnp.int32),
        pltpu.SemaphoreType.DMA,
    ],
)
def _scatter_partials(src_hbm, idx_hbm, part_hbm, acc, rows_v, idx_v, sem):
    cid = lax.axis_index("c")
    sid = lax.axis_index("s")
    wid = cid * NS + sid

    @pl.loop(0, CHUNK * FEAT // 16)
    def _(i):
        r = i // (FEAT // 16)
        col = i % (FEAT // 16)
        rows_v[r, pl.ds(col * 16, 16)] = jnp.zeros((16,), jnp.float32)

    base = sid * ROWS_A

    def _tile_rows():
        yield (sid < NS - 1, _split(ROWS_A))
        yield (sid == NS - 1, _split(ROWS_LAST))

    for cond, pieces in _tile_rows():
        @pl.when(cond)
        def _(pieces=pieces):
            off = 0
            for n in pieces:
                pltpu.sync_copy(rows_v.at[pl.ds(0, n)],
                                acc.at[pl.ds(base + off, n)])
                off += n
    plsc.subcore_barrier()

    ebase = wid * EDGES_PER_W

    @pl.loop(0, NCHUNKS)
    def _(g):
        pltpu.sync_copy(src_hbm.at[pl.ds(ebase + g * CHUNK, CHUNK)], rows_v)
        pltpu.sync_copy(idx_hbm.at[wid * NCHUNKS + g], idx_v)
        for k in range(KSUB):
            pltpu.async_copy(rows_v.at[pl.ds(k * SUB, SUB)],
                             acc.at[idx_v.at[k]], sem, add=True).wait()

    plsc.subcore_barrier()

    for cond, pieces in _tile_rows():
        @pl.when(cond)
        def _(pieces=pieces):
            off = 0
            for n in pieces:
                pltpu.sync_copy(acc.at[pl.ds(base + off, n)],
                                rows_v.at[pl.ds(0, n)])
                pltpu.sync_copy(rows_v.at[pl.ds(0, n)],
                                part_hbm.at[cid].at[pl.ds(base + off, n)])
                off += n


def _combine(parts):
    def body(p_ref, o_ref):
        o_ref[...] = p_ref[0] + p_ref[1]

    return pl.pallas_call(
        body,
        grid=(10,),
        in_specs=[pl.BlockSpec((NC, NUM_NODES // 10, FEAT), lambda i: (0, i, 0))],
        out_specs=pl.BlockSpec((NUM_NODES // 10, FEAT), lambda i: (i, 0)),
        out_shape=jax.ShapeDtypeStruct((NUM_NODES, FEAT), jnp.float32),
    )(parts)


@jax.jit
def kernel(src, indices):
    idx3 = indices.astype(jnp.int32).reshape(NW * NCHUNKS, KSUB, SUB)
    parts = _scatter_partials(src, idx3)
    return _combine(parts)


# speedup vs baseline: 7.2469x; 1.4804x over previous
import functools

import jax
import jax.numpy as jnp
from jax import lax
from jax.experimental import pallas as pl
from jax.experimental.pallas import tpu as pltpu
from jax.experimental.pallas import tpu_sc as plsc

NUM_NODES = 10000
NUM_EDGES = 320000
FEAT = 128

NC = 2
NS = 16
NW = NC * NS

EDGES_PER_W = NUM_EDGES // NW
CHUNK = 80
NCHUNKS = EDGES_PER_W // CHUNK
SUB = 80
KSUB = CHUNK // SUB
ROWS_A = 632
ROWS_LAST = NUM_NODES - ROWS_A * (NS - 1)

_mesh = plsc.VectorSubcoreMesh(core_axis_name="c", subcore_axis_name="s")


def _split(count):
    out = [CHUNK] * (count // CHUNK)
    if count % CHUNK:
        out.append(count % CHUNK)
    return out


@functools.partial(
    pl.kernel,
    out_type=jax.ShapeDtypeStruct((NC, NUM_NODES, FEAT), jnp.float32),
    mesh=_mesh,
    scratch_types=[
        pltpu.VMEM_SHARED((NUM_NODES, FEAT), jnp.float32),
        pltpu.VMEM((2, CHUNK, FEAT), jnp.float32),
        pltpu.VMEM((2, KSUB, SUB), jnp.int32),
        pltpu.SemaphoreType.DMA,
        pltpu.SemaphoreType.DMA,
        pltpu.SemaphoreType.DMA,
        pltpu.SemaphoreType.DMA,
        pltpu.SemaphoreType.DMA,
    ],
)
def _scatter_partials(src_hbm, idx_hbm, part_hbm, acc, rows_v, idx_v,
                      rsem0, rsem1, isem0, isem1, ssem):
    cid = lax.axis_index("c")
    sid = lax.axis_index("s")
    wid = cid * NS + sid

    @pl.loop(0, CHUNK * FEAT // 16)
    def _(i):
        r = i // (FEAT // 16)
        col = i % (FEAT // 16)
        rows_v.at[0][r, pl.ds(col * 16, 16)] = jnp.zeros((16,), jnp.float32)

    base = sid * ROWS_A

    def _tile_rows():
        yield (sid < NS - 1, _split(ROWS_A))
        yield (sid == NS - 1, _split(ROWS_LAST))

    for cond, pieces in _tile_rows():
        @pl.when(cond)
        def _(pieces=pieces):
            off = 0
            for n in pieces:
                pltpu.sync_copy(rows_v.at[0].at[pl.ds(0, n)],
                                acc.at[pl.ds(base + off, n)])
                off += n
    plsc.subcore_barrier()

    ebase = wid * EDGES_PER_W

    rsems = (rsem0, rsem1)
    isems = (isem0, isem1)

    def fill_start(g, b):
        pltpu.async_copy(src_hbm.at[pl.ds(ebase + g * CHUNK, CHUNK)],
                         rows_v.at[b], rsems[b])
        pltpu.async_copy(idx_hbm.at[wid * NCHUNKS + g], idx_v.at[b],
                         isems[b])

    def fill_wait(g, b):
        pltpu.make_async_copy(src_hbm.at[pl.ds(ebase + g * CHUNK, CHUNK)],
                              rows_v.at[b], rsems[b]).wait()
        pltpu.make_async_copy(idx_hbm.at[wid * NCHUNKS + g], idx_v.at[b],
                              isems[b]).wait()

    def scatter(b):
        descs = [
            pltpu.async_copy(rows_v.at[b].at[pl.ds(k * SUB, SUB)],
                             acc.at[idx_v.at[b].at[k]], ssem, add=True)
            for k in range(KSUB)
        ]
        for d in descs:
            d.wait()

    fill_start(0, 0)
    fill_start(1, 1)

    @pl.loop(0, NCHUNKS - 3, step=2)
    def _(g):
        for b in range(2):
            fill_wait(g + b, b)
            scatter(b)
            fill_start(g + 2 + b, b)

    fill_wait(NCHUNKS - 3, 0)
    scatter(0)
    fill_start(NCHUNKS - 1, 0)
    fill_wait(NCHUNKS - 2, 1)
    scatter(1)
    fill_wait(NCHUNKS - 1, 0)
    scatter(0)

    plsc.subcore_barrier()

    for cond, pieces in _tile_rows():
        @pl.when(cond)
        def _(pieces=pieces):
            off = 0
            for n in pieces:
                pltpu.sync_copy(acc.at[pl.ds(base + off, n)],
                                rows_v.at[0].at[pl.ds(0, n)])
                pltpu.sync_copy(rows_v.at[0].at[pl.ds(0, n)],
                                part_hbm.at[cid].at[pl.ds(base + off, n)])
                off += n


def _combine(parts):
    def body(p_ref, o_ref):
        o_ref[...] = p_ref[0] + p_ref[1]

    return pl.pallas_call(
        body,
        grid=(10,),
        in_specs=[pl.BlockSpec((NC, NUM_NODES // 10, FEAT), lambda i: (0, i, 0))],
        out_specs=pl.BlockSpec((NUM_NODES // 10, FEAT), lambda i: (i, 0)),
        out_shape=jax.ShapeDtypeStruct((NUM_NODES, FEAT), jnp.float32),
    )(parts)


@jax.jit
def kernel(src, indices):
    idx3 = indices.astype(jnp.int32).reshape(NW * NCHUNKS, KSUB, SUB)
    parts = _scatter_partials(src, idx3)
    return _combine(parts)


# speedup vs baseline: 8.0333x; 1.1085x over previous
import functools

import jax
import jax.numpy as jnp
from jax import lax
from jax.experimental import pallas as pl
from jax.experimental.pallas import tpu as pltpu
from jax.experimental.pallas import tpu_sc as plsc

NUM_NODES = 10000
NUM_EDGES = 320000
FEAT = 128

NC = 2
NS = 16
NW = NC * NS

EDGES_PER_W = NUM_EDGES // NW
CHUNK = 80
NCHUNKS = EDGES_PER_W // CHUNK
SUB = 80
KSUB = CHUNK // SUB
NBUF = 4
ROWS_A = 632
ROWS_LAST = NUM_NODES - ROWS_A * (NS - 1)

_mesh = plsc.VectorSubcoreMesh(core_axis_name="c", subcore_axis_name="s")


def _split(count):
    out = [CHUNK] * (count // CHUNK)
    if count % CHUNK:
        out.append(count % CHUNK)
    return out


@functools.partial(
    pl.kernel,
    out_type=jax.ShapeDtypeStruct((NC, NUM_NODES, FEAT), jnp.float32),
    mesh=_mesh,
    scratch_types=[
        pltpu.VMEM_SHARED((NUM_NODES, FEAT), jnp.float32),
        pltpu.VMEM((NBUF, CHUNK, FEAT), jnp.float32),
        pltpu.VMEM((NBUF, KSUB, SUB), jnp.int32),
    ] + [pltpu.SemaphoreType.DMA] * (2 * NBUF + 1),
)
def _scatter_partials(src_hbm, idx_hbm, part_hbm, acc, rows_v, idx_v, *sems):
    cid = lax.axis_index("c")
    sid = lax.axis_index("s")
    wid = cid * NS + sid

    @pl.loop(0, CHUNK * FEAT // 16)
    def _(i):
        r = i // (FEAT // 16)
        col = i % (FEAT // 16)
        rows_v.at[0][r, pl.ds(col * 16, 16)] = jnp.zeros((16,), jnp.float32)

    base = sid * ROWS_A

    def _tile_rows():
        yield (sid < NS - 1, _split(ROWS_A))
        yield (sid == NS - 1, _split(ROWS_LAST))

    for cond, pieces in _tile_rows():
        @pl.when(cond)
        def _(pieces=pieces):
            off = 0
            for n in pieces:
                pltpu.sync_copy(rows_v.at[0].at[pl.ds(0, n)],
                                acc.at[pl.ds(base + off, n)])
                off += n
    plsc.subcore_barrier()

    ebase = wid * EDGES_PER_W

    rsems = sems[:NBUF]
    isems = sems[NBUF:2 * NBUF]
    ssem = sems[2 * NBUF]

    def fill_start(g, b):
        pltpu.async_copy(src_hbm.at[pl.ds(ebase + g * CHUNK, CHUNK)],
                         rows_v.at[b], rsems[b])
        pltpu.async_copy(idx_hbm.at[wid * NCHUNKS + g], idx_v.at[b],
                         isems[b])

    def fill_wait(g, b):
        pltpu.make_async_copy(src_hbm.at[pl.ds(ebase + g * CHUNK, CHUNK)],
                              rows_v.at[b], rsems[b]).wait()
        pltpu.make_async_copy(idx_hbm.at[wid * NCHUNKS + g], idx_v.at[b],
                              isems[b]).wait()

    def scatter(b):
        descs = [
            pltpu.async_copy(rows_v.at[b].at[pl.ds(k * SUB, SUB)],
                             acc.at[idx_v.at[b].at[k]], ssem, add=True)
            for k in range(KSUB)
        ]
        for d in descs:
            d.wait()

    for b in range(NBUF):
        fill_start(b, b)

    LOOPEND = ((NCHUNKS - NBUF) // NBUF) * NBUF

    @pl.loop(0, LOOPEND, step=NBUF)
    def _(g):
        for b in range(NBUF):
            fill_wait(g + b, b)
            scatter(b)
            fill_start(g + NBUF + b, b)

    for ch in range(LOOPEND, NCHUNKS):
        b = ch % NBUF
        fill_wait(ch, b)
        scatter(b)
        if ch + NBUF < NCHUNKS:
            fill_start(ch + NBUF, b)

    plsc.subcore_barrier()

    for cond, pieces in _tile_rows():
        @pl.when(cond)
        def _(pieces=pieces):
            off = 0
            for n in pieces:
                pltpu.sync_copy(acc.at[pl.ds(base + off, n)],
                                rows_v.at[0].at[pl.ds(0, n)])
                pltpu.sync_copy(rows_v.at[0].at[pl.ds(0, n)],
                                part_hbm.at[cid].at[pl.ds(base + off, n)])
                off += n


def _combine(parts):
    def body(p_ref, o_ref):
        o_ref[...] = p_ref[0] + p_ref[1]

    return pl.pallas_call(
        body,
        grid=(10,),
        in_specs=[pl.BlockSpec((NC, NUM_NODES // 10, FEAT), lambda i: (0, i, 0))],
        out_specs=pl.BlockSpec((NUM_NODES // 10, FEAT), lambda i: (i, 0)),
        out_shape=jax.ShapeDtypeStruct((NUM_NODES, FEAT), jnp.float32),
    )(parts)


@jax.jit
def kernel(src, indices):
    idx3 = indices.astype(jnp.int32).reshape(NW * NCHUNKS, KSUB, SUB)
    parts = _scatter_partials(src, idx3)
    return _combine(parts)
